# baseline (speedup 1.0000x reference)
import functools

import jax
import jax.numpy as jnp
from jax import lax
from jax.experimental import pallas as pl
from jax.experimental.pallas import tpu as pltpu
from jax.experimental.pallas import tpu_sc as plsc

NC = 2
NS = 16
LANES = 16
CHUNK = 128


def _pad_nodes(n):
  return ((n + NS * CHUNK - 1) // (NS * CHUNK)) * (NS * CHUNK)


def _sc_degree(cols3d, n_nodes, rounds):
  n_pad = _pad_nodes(n_nodes)
  rows_per_sub = n_pad // NS
  mesh = plsc.VectorSubcoreMesh(core_axis_name="c", subcore_axis_name="s", num_cores=NC, num_subcores=NS)

  @functools.partial(
      pl.kernel,
      out_type=jax.ShapeDtypeStruct((NC, n_pad, LANES), jnp.float32),
      mesh=mesh,
      scratch_types=[
          pltpu.VMEM((rounds, CHUNK), jnp.int32),
          pltpu.VMEM((CHUNK, LANES), jnp.float32),
          pltpu.VMEM((rows_per_sub, LANES), jnp.float32),
          pltpu.VMEM_SHARED((n_pad, LANES), jnp.float32),
      ],
      compiler_params=pltpu.CompilerParams(use_tc_tiling_on_sc=False),
  )
  def deg_kernel(cols_hbm, out_hbm, idx_v, ones_v, zero_v, acc_sh):
    c = lax.axis_index("c")
    s = lax.axis_index("s")
    wid = s * NC + c

    zero16 = jnp.zeros((LANES,), jnp.float32)
    one_row = jnp.where(lax.iota(jnp.int32, LANES) == 0, 1.0, 0.0)

    @pl.loop(0, rows_per_sub)
    def _(r):
      zero_v[r, :] = zero16

    @pl.loop(0, CHUNK)
    def _(r):
      ones_v[r, :] = one_row

    pltpu.sync_copy(zero_v, acc_sh.at[pl.ds(s * rows_per_sub, rows_per_sub)])
    pltpu.sync_copy(cols_hbm.at[wid], idx_v)
    plsc.subcore_barrier()

    @pl.loop(0, rounds)
    def _(k):
      pltpu.sync_copy(ones_v, acc_sh.at[idx_v.at[k]], add=True)

    plsc.subcore_barrier()
    pltpu.sync_copy(
        acc_sh.at[pl.ds(s * rows_per_sub, rows_per_sub)],
        out_hbm.at[c, pl.ds(s * rows_per_sub, rows_per_sub)],
    )

  return deg_kernel(cols3d)


def _sc_spmm(h2, src3d, dst3d, n_nodes, rounds):
  d = h2.shape[1]
  n_pad = _pad_nodes(n_nodes)
  rows_per_sub = n_pad // NS
  zrows = CHUNK
  mesh = plsc.VectorSubcoreMesh(core_axis_name="c", subcore_axis_name="s", num_cores=NC, num_subcores=NS)

  @functools.partial(
      pl.kernel,
      out_type=jax.ShapeDtypeStruct((NC, n_pad, d), jnp.float32),
      mesh=mesh,
      scratch_types=[
          pltpu.VMEM((CHUNK,), jnp.int32),
          pltpu.VMEM((CHUNK,), jnp.int32),
          pltpu.VMEM((CHUNK, d), jnp.float32),
          pltpu.VMEM_SHARED((n_pad, d), jnp.float32),
      ],
  )
  def spmm_kernel(h2_hbm, src_hbm, dst_hbm, out_hbm, sidx_s, didx_s, gbuf0,
                  acc_sh):
    c = lax.axis_index("c")
    s = lax.axis_index("s")
    wid = s * NC + c

    zero16 = jnp.zeros((LANES,), jnp.float32)

    @pl.loop(0, zrows)
    def _(r):
      @pl.loop(0, d, step=LANES)
      def _(j):
        gbuf0[r, pl.ds(j, LANES)] = zero16

    @pl.loop(0, rows_per_sub, step=zrows)
    def _(r0):
      pltpu.sync_copy(
          gbuf0.at[pl.ds(0, zrows)],
          acc_sh.at[pl.ds(s * rows_per_sub + r0, zrows)],
      )

    plsc.subcore_barrier()

    @pl.loop(0, rounds)
    def _(k):
      pltpu.sync_copy(src_hbm.at[wid, k], sidx_s)
      pltpu.sync_copy(dst_hbm.at[wid, k], didx_s)
      pltpu.sync_copy(h2_hbm.at[sidx_s], gbuf0)
      pltpu.sync_copy(gbuf0, acc_sh.at[didx_s], add=True)

    plsc.subcore_barrier()
    pltpu.sync_copy(
        acc_sh.at[pl.ds(s * rows_per_sub, rows_per_sub)],
        out_hbm.at[c, pl.ds(s * rows_per_sub, rows_per_sub)],
    )

  return spmm_kernel(h2, src3d, dst3d)


def _tc_linear(x, w):
  n, d_in = x.shape
  d_out = w.shape[0]
  bm = 1000

  def body(x_ref, w_ref, o_ref):
    o_ref[...] = lax.dot_general(
        x_ref[...], w_ref[...],
        (((1,), (1,)), ((), ())),
        precision=lax.Precision.HIGHEST,
    )

  return pl.pallas_call(
      body,
      grid=(n // bm,),
      in_specs=[
          pl.BlockSpec((bm, d_in), lambda i: (i, 0)),
          pl.BlockSpec((d_out, d_in), lambda i: (0, 0)),
      ],
      out_specs=pl.BlockSpec((bm, d_out), lambda i: (i, 0)),
      out_shape=jax.ShapeDtypeStruct((n, d_out), jnp.float32),
  )(x, w)


def _tc_scale(h, degp):
  n, d = h.shape
  bm = 1000

  def body(h_ref, dp_ref, o_ref):
    deg = 1.0 + dp_ref[0, :, 0] + dp_ref[1, :, 0]
    o_ref[...] = h_ref[...] * lax.rsqrt(deg)[:, None]

  return pl.pallas_call(
      body,
      grid=(n // bm,),
      in_specs=[
          pl.BlockSpec((bm, d), lambda i: (i, 0)),
          pl.BlockSpec((NC, bm, LANES), lambda i: (0, i, 0)),
      ],
      out_specs=pl.BlockSpec((bm, d), lambda i: (i, 0)),
      out_shape=jax.ShapeDtypeStruct((n, d), jnp.float32),
  )(h, degp)


def _tc_combine(q, degp, h2):
  n, d = h2.shape
  bm = 1000

  def body(q_ref, dp_ref, h2_ref, o_ref):
    deg = 1.0 + dp_ref[0, :, 0] + dp_ref[1, :, 0]
    agg = q_ref[0] + q_ref[1] + h2_ref[...]
    o_ref[...] = jnp.maximum(agg * lax.rsqrt(deg)[:, None], 0.0)

  return pl.pallas_call(
      body,
      grid=(n // bm,),
      in_specs=[
          pl.BlockSpec((NC, bm, d), lambda i: (0, i, 0)),
          pl.BlockSpec((NC, bm, LANES), lambda i: (0, i, 0)),
          pl.BlockSpec((bm, d), lambda i: (i, 0)),
      ],
      out_specs=pl.BlockSpec((bm, d), lambda i: (i, 0)),
      out_shape=jax.ShapeDtypeStruct((n, d), jnp.float32),
  )(q, degp, h2)


def kernel(x, edge_index, W):
  n = x.shape[0]
  n_pad = _pad_nodes(n)
  garbage_row = n_pad - 8
  ei = edge_index.astype(jnp.int32)
  e = ei.shape[1]
  n_chunks = e // CHUNK
  nw = NC * NS

  rounds = -(-n_chunks // nw)
  rounds = ((rounds + 3) // 4) * 4
  pad = rounds * nw * CHUNK - e

  def to_worker_blocks(flat, pad_values):
    padded = jnp.concatenate([flat, pad_values])
    return padded.reshape(rounds, nw, CHUNK).transpose(1, 0, 2)

  pad_zero = jnp.zeros((pad,), jnp.int32)
  pad_spread = n + (jnp.arange(pad, dtype=jnp.int32) % (n_pad - n - 8))
  src3d = to_worker_blocks(ei[1], pad_zero)
  dst3d = to_worker_blocks(ei[0], pad_spread)
  cols3d = to_worker_blocks(ei[1], pad_spread)

  degp = _sc_degree(cols3d, n, rounds)[:, :n]
  h = _tc_linear(x, W)
  h2 = _tc_scale(h, degp)
  q = _sc_spmm(h2, src3d, dst3d, n, rounds)[:, :n]
  return _tc_combine(q, degp, h2)


# speedup vs baseline: 1.8116x; 1.8116x over previous
import functools

import jax
import jax.numpy as jnp
from jax import lax
from jax.experimental import pallas as pl
from jax.experimental.pallas import tpu as pltpu
from jax.experimental.pallas import tpu_sc as plsc

NC = 2
NS = 16
LANES = 16
CHUNK = 128


def _pad_nodes(n):
  return ((n + NS * CHUNK - 1) // (NS * CHUNK)) * (NS * CHUNK)


def _sc_degree(cols3d, n_nodes, rounds):
  n_pad = _pad_nodes(n_nodes)
  rows_per_sub = n_pad // NS
  mesh = plsc.VectorSubcoreMesh(core_axis_name="c", subcore_axis_name="s", num_cores=NC, num_subcores=NS)

  @functools.partial(
      pl.kernel,
      out_type=jax.ShapeDtypeStruct((NC, n_pad, LANES), jnp.float32),
      mesh=mesh,
      scratch_types=[
          pltpu.VMEM((rounds, CHUNK), jnp.int32),
          pltpu.VMEM((CHUNK, LANES), jnp.float32),
          pltpu.VMEM((rows_per_sub, LANES), jnp.float32),
          pltpu.VMEM_SHARED((n_pad, LANES), jnp.float32),
      ],
      compiler_params=pltpu.CompilerParams(use_tc_tiling_on_sc=False),
  )
  def deg_kernel(cols_hbm, out_hbm, idx_v, ones_v, zero_v, acc_sh):
    c = lax.axis_index("c")
    s = lax.axis_index("s")
    wid = s * NC + c

    zero16 = jnp.zeros((LANES,), jnp.float32)
    one_row = jnp.where(lax.iota(jnp.int32, LANES) == 0, 1.0, 0.0)

    @pl.loop(0, rows_per_sub)
    def _(r):
      zero_v[r, :] = zero16

    @pl.loop(0, CHUNK)
    def _(r):
      ones_v[r, :] = one_row

    pltpu.sync_copy(zero_v, acc_sh.at[pl.ds(s * rows_per_sub, rows_per_sub)])
    pltpu.sync_copy(cols_hbm.at[wid], idx_v)
    plsc.subcore_barrier()

    @pl.loop(0, rounds)
    def _(k):
      pltpu.sync_copy(ones_v, acc_sh.at[idx_v.at[k]], add=True)

    plsc.subcore_barrier()
    pltpu.sync_copy(
        acc_sh.at[pl.ds(s * rows_per_sub, rows_per_sub)],
        out_hbm.at[c, pl.ds(s * rows_per_sub, rows_per_sub)],
    )

  return deg_kernel(cols3d)


def _sc_spmm(h2, src, dst, n_nodes):
  e = src.shape[0]
  n_chunks = e // CHUNK
  d = h2.shape[1]
  n_pad = _pad_nodes(n_nodes)
  rows_per_sub = n_pad // NS
  zrows = CHUNK
  mesh = plsc.VectorSubcoreMesh(core_axis_name="c", subcore_axis_name="s", num_cores=NC, num_subcores=NS)

  @functools.partial(
      pl.kernel,
      out_type=jax.ShapeDtypeStruct((NC, n_pad, d), jnp.float32),
      mesh=mesh,
      scratch_types=[
          pltpu.VMEM((CHUNK,), jnp.int32),
          pltpu.VMEM((CHUNK,), jnp.int32),
          pltpu.VMEM((CHUNK, d), jnp.float32),
          pltpu.VMEM_SHARED((n_pad, d), jnp.float32),
      ],
  )
  def spmm_kernel(h2_hbm, src_hbm, dst_hbm, out_hbm, sidx_s, didx_s, gbuf0,
                  acc_sh):
    c = lax.axis_index("c")
    s = lax.axis_index("s")
    wid = s * NC + c

    zero16 = jnp.zeros((LANES,), jnp.float32)

    @pl.loop(0, zrows)
    def _(r):
      @pl.loop(0, d, step=LANES)
      def _(j):
        gbuf0[r, pl.ds(j, LANES)] = zero16

    @pl.loop(0, rows_per_sub, step=zrows)
    def _(r0):
      pltpu.sync_copy(
          gbuf0.at[pl.ds(0, zrows)],
          acc_sh.at[pl.ds(s * rows_per_sub + r0, zrows)],
      )

    plsc.subcore_barrier()

    max_rounds = (n_chunks + NC * NS - 1) // (NC * NS)

    @pl.loop(0, max_rounds)
    def _(k):
      g = wid + k * (NC * NS)

      @pl.when(g < n_chunks)
      def _():
        pltpu.sync_copy(src_hbm.at[pl.ds(g * CHUNK, CHUNK)], sidx_s)
        pltpu.sync_copy(dst_hbm.at[pl.ds(g * CHUNK, CHUNK)], didx_s)
        pltpu.sync_copy(h2_hbm.at[sidx_s], gbuf0)
        pltpu.sync_copy(gbuf0, acc_sh.at[didx_s], add=True)

    plsc.subcore_barrier()
    pltpu.sync_copy(
        acc_sh.at[pl.ds(s * rows_per_sub, rows_per_sub)],
        out_hbm.at[c, pl.ds(s * rows_per_sub, rows_per_sub)],
    )

  return spmm_kernel(h2, src, dst)


def _tc_linear(x, w):
  n, d_in = x.shape
  d_out = w.shape[0]
  bm = 1000

  def body(x_ref, w_ref, o_ref):
    o_ref[...] = lax.dot_general(
        x_ref[...], w_ref[...],
        (((1,), (1,)), ((), ())),
        precision=lax.Precision.HIGHEST,
    )

  return pl.pallas_call(
      body,
      grid=(n // bm,),
      in_specs=[
          pl.BlockSpec((bm, d_in), lambda i: (i, 0)),
          pl.BlockSpec((d_out, d_in), lambda i: (0, 0)),
      ],
      out_specs=pl.BlockSpec((bm, d_out), lambda i: (i, 0)),
      out_shape=jax.ShapeDtypeStruct((n, d_out), jnp.float32),
  )(x, w)


def _tc_scale(h, degp):
  n, d = h.shape
  bm = 1000

  def body(h_ref, dp_ref, o_ref):
    deg = 1.0 + dp_ref[0, :, 0] + dp_ref[1, :, 0]
    o_ref[...] = h_ref[...] * lax.rsqrt(deg)[:, None]

  return pl.pallas_call(
      body,
      grid=(n // bm,),
      in_specs=[
          pl.BlockSpec((bm, d), lambda i: (i, 0)),
          pl.BlockSpec((NC, bm, LANES), lambda i: (0, i, 0)),
      ],
      out_specs=pl.BlockSpec((bm, d), lambda i: (i, 0)),
      out_shape=jax.ShapeDtypeStruct((n, d), jnp.float32),
  )(h, degp)


def _tc_combine(q, degp, h2):
  n, d = h2.shape
  bm = 1000

  def body(q_ref, dp_ref, h2_ref, o_ref):
    deg = 1.0 + dp_ref[0, :, 0] + dp_ref[1, :, 0]
    agg = q_ref[0] + q_ref[1] + h2_ref[...]
    o_ref[...] = jnp.maximum(agg * lax.rsqrt(deg)[:, None], 0.0)

  return pl.pallas_call(
      body,
      grid=(n // bm,),
      in_specs=[
          pl.BlockSpec((NC, bm, d), lambda i: (0, i, 0)),
          pl.BlockSpec((NC, bm, LANES), lambda i: (0, i, 0)),
          pl.BlockSpec((bm, d), lambda i: (i, 0)),
      ],
      out_specs=pl.BlockSpec((bm, d), lambda i: (i, 0)),
      out_shape=jax.ShapeDtypeStruct((n, d), jnp.float32),
  )(q, degp, h2)


def kernel(x, edge_index, W):
  n = x.shape[0]
  n_pad = _pad_nodes(n)
  garbage_row = n_pad - 8
  ei = edge_index.astype(jnp.int32)
  e = ei.shape[1]
  n_chunks = e // CHUNK
  nw = NC * NS

  rounds = -(-n_chunks // nw)
  rounds = ((rounds + 3) // 4) * 4
  pad = rounds * nw * CHUNK - e

  def to_worker_blocks(flat, pad_values):
    padded = jnp.concatenate([flat, pad_values])
    return padded.reshape(rounds, nw, CHUNK).transpose(1, 0, 2)

  pad_zero = jnp.zeros((pad,), jnp.int32)
  pad_spread = n + (jnp.arange(pad, dtype=jnp.int32) % (n_pad - n - 8))
  cols3d = to_worker_blocks(ei[1], pad_spread)

  degp = _sc_degree(cols3d, n, rounds)[:, :n]
  h = _tc_linear(x, W)
  h2 = _tc_scale(h, degp)
  q = _sc_spmm(h2, ei[1], ei[0], n)[:, :n]
  return _tc_combine(q, degp, h2)


# speedup vs baseline: 2.5408x; 1.4025x over previous
import functools

import jax
import jax.numpy as jnp
from jax import lax
from jax.experimental import pallas as pl
from jax.experimental.pallas import tpu as pltpu
from jax.experimental.pallas import tpu_sc as plsc

NC = 2
NS = 16
LANES = 16
CHUNK = 128


def _pad_nodes(n):
  return ((n + NS * CHUNK - 1) // (NS * CHUNK)) * (NS * CHUNK)


def _sc_degree(cols3d, n_nodes, rounds):
  n_pad = _pad_nodes(n_nodes)
  rows_per_sub = n_pad // NS
  mesh = plsc.VectorSubcoreMesh(core_axis_name="c", subcore_axis_name="s", num_cores=NC, num_subcores=NS)

  @functools.partial(
      pl.kernel,
      out_type=jax.ShapeDtypeStruct((NC, n_pad, LANES), jnp.float32),
      mesh=mesh,
      scratch_types=[
          pltpu.VMEM((rounds, CHUNK), jnp.int32),
          pltpu.VMEM((CHUNK, LANES), jnp.float32),
          pltpu.VMEM((rows_per_sub, LANES), jnp.float32),
          pltpu.VMEM_SHARED((n_pad, LANES), jnp.float32),
      ],
      compiler_params=pltpu.CompilerParams(use_tc_tiling_on_sc=False),
  )
  def deg_kernel(cols_hbm, out_hbm, idx_v, ones_v, zero_v, acc_sh):
    c = lax.axis_index("c")
    s = lax.axis_index("s")
    wid = s * NC + c

    zero16 = jnp.zeros((LANES,), jnp.float32)
    one_row = jnp.where(lax.iota(jnp.int32, LANES) == 0, 1.0, 0.0)

    @pl.loop(0, rows_per_sub)
    def _(r):
      zero_v[r, :] = zero16

    @pl.loop(0, CHUNK)
    def _(r):
      ones_v[r, :] = one_row

    pltpu.sync_copy(zero_v, acc_sh.at[pl.ds(s * rows_per_sub, rows_per_sub)])
    pltpu.sync_copy(cols_hbm.at[wid], idx_v)
    plsc.subcore_barrier()

    @pl.loop(0, rounds)
    def _(k):
      pltpu.sync_copy(ones_v, acc_sh.at[idx_v.at[k]], add=True)

    plsc.subcore_barrier()
    pltpu.sync_copy(
        acc_sh.at[pl.ds(s * rows_per_sub, rows_per_sub)],
        out_hbm.at[c, pl.ds(s * rows_per_sub, rows_per_sub)],
    )

  return deg_kernel(cols3d)


def _sc_spmm(h2, src, dst, n_nodes):
  e = src.shape[0]
  n_chunks = e // CHUNK
  d = h2.shape[1]
  n_pad = _pad_nodes(n_nodes)
  rows_per_sub = n_pad // NS
  zrows = CHUNK
  mesh = plsc.VectorSubcoreMesh(core_axis_name="c", subcore_axis_name="s", num_cores=NC, num_subcores=NS)

  @functools.partial(
      pl.kernel,
      out_type=jax.ShapeDtypeStruct((NC, n_pad, d), jnp.float32),
      mesh=mesh,
      scratch_types=[
          pltpu.VMEM((CHUNK,), jnp.int32),
          pltpu.VMEM((CHUNK,), jnp.int32),
          pltpu.VMEM((CHUNK,), jnp.int32),
          pltpu.VMEM((CHUNK,), jnp.int32),
          pltpu.VMEM((CHUNK, d), jnp.float32),
          pltpu.VMEM((CHUNK, d), jnp.float32),
          pltpu.VMEM_SHARED((n_pad, d), jnp.float32),
          pltpu.SemaphoreType.DMA,
          pltpu.SemaphoreType.DMA,
      ],
  )
  def spmm_kernel(h2_hbm, src_hbm, dst_hbm, out_hbm, sidx0, didx0, sidx1,
                  didx1, gbuf0, gbuf1, acc_sh, sem0, sem1):
    c = lax.axis_index("c")
    s = lax.axis_index("s")
    wid = s * NC + c

    zero16 = jnp.zeros((LANES,), jnp.float32)

    @pl.loop(0, zrows)
    def _(r):
      @pl.loop(0, d, step=LANES)
      def _(j):
        gbuf0[r, pl.ds(j, LANES)] = zero16

    @pl.loop(0, rows_per_sub, step=zrows)
    def _(r0):
      pltpu.sync_copy(
          gbuf0.at[pl.ds(0, zrows)],
          acc_sh.at[pl.ds(s * rows_per_sub + r0, zrows)],
      )

    plsc.subcore_barrier()

    nw = NC * NS
    max_rounds = (n_chunks + nw - 1) // nw
    max_rounds += max_rounds % 2

    def fetch(g, sidx, didx, gbuf, sem):
      pltpu.sync_copy(src_hbm.at[pl.ds(g * CHUNK, CHUNK)], sidx)
      pltpu.sync_copy(dst_hbm.at[pl.ds(g * CHUNK, CHUNK)], didx)
      pltpu.make_async_copy(h2_hbm.at[sidx], gbuf, sem).start()

    def drain(didx, gbuf, sem):
      pltpu.make_async_copy(h2_hbm.at[sidx0], gbuf, sem).wait()
      pltpu.sync_copy(gbuf, acc_sh.at[didx], add=True)

    @pl.when(wid < n_chunks)
    def _():
      fetch(wid, sidx0, didx0, gbuf0, sem0)

    @pl.loop(0, max_rounds, step=2)
    def _(k):
      g0 = wid + k * nw
      g1 = g0 + nw
      g2 = g1 + nw

      @pl.when(g1 < n_chunks)
      def _():
        fetch(g1, sidx1, didx1, gbuf1, sem1)

      @pl.when(g0 < n_chunks)
      def _():
        drain(didx0, gbuf0, sem0)

      @pl.when(g2 < n_chunks)
      def _():
        fetch(g2, sidx0, didx0, gbuf0, sem0)

      @pl.when(g1 < n_chunks)
      def _():
        drain(didx1, gbuf1, sem1)

    plsc.subcore_barrier()
    pltpu.sync_copy(
        acc_sh.at[pl.ds(s * rows_per_sub, rows_per_sub)],
        out_hbm.at[c, pl.ds(s * rows_per_sub, rows_per_sub)],
    )

  return spmm_kernel(h2, src, dst)


def _tc_linear(x, w):
  n, d_in = x.shape
  d_out = w.shape[0]
  bm = 1000

  def body(x_ref, w_ref, o_ref):
    o_ref[...] = lax.dot_general(
        x_ref[...], w_ref[...],
        (((1,), (1,)), ((), ())),
        precision=lax.Precision.HIGHEST,
    )

  return pl.pallas_call(
      body,
      grid=(n // bm,),
      in_specs=[
          pl.BlockSpec((bm, d_in), lambda i: (i, 0)),
          pl.BlockSpec((d_out, d_in), lambda i: (0, 0)),
      ],
      out_specs=pl.BlockSpec((bm, d_out), lambda i: (i, 0)),
      out_shape=jax.ShapeDtypeStruct((n, d_out), jnp.float32),
  )(x, w)


def _tc_scale(h, degp):
  n, d = h.shape
  bm = 1000

  def body(h_ref, dp_ref, o_ref):
    deg = 1.0 + dp_ref[0, :, 0] + dp_ref[1, :, 0]
    o_ref[...] = h_ref[...] * lax.rsqrt(deg)[:, None]

  return pl.pallas_call(
      body,
      grid=(n // bm,),
      in_specs=[
          pl.BlockSpec((bm, d), lambda i: (i, 0)),
          pl.BlockSpec((NC, bm, LANES), lambda i: (0, i, 0)),
      ],
      out_specs=pl.BlockSpec((bm, d), lambda i: (i, 0)),
      out_shape=jax.ShapeDtypeStruct((n, d), jnp.float32),
  )(h, degp)


def _tc_combine(q, degp, h2):
  n, d = h2.shape
  bm = 1000

  def body(q_ref, dp_ref, h2_ref, o_ref):
    deg = 1.0 + dp_ref[0, :, 0] + dp_ref[1, :, 0]
    agg = q_ref[0] + q_ref[1] + h2_ref[...]
    o_ref[...] = jnp.maximum(agg * lax.rsqrt(deg)[:, None], 0.0)

  return pl.pallas_call(
      body,
      grid=(n // bm,),
      in_specs=[
          pl.BlockSpec((NC, bm, d), lambda i: (0, i, 0)),
          pl.BlockSpec((NC, bm, LANES), lambda i: (0, i, 0)),
          pl.BlockSpec((bm, d), lambda i: (i, 0)),
      ],
      out_specs=pl.BlockSpec((bm, d), lambda i: (i, 0)),
      out_shape=jax.ShapeDtypeStruct((n, d), jnp.float32),
  )(q, degp, h2)


def kernel(x, edge_index, W):
  n = x.shape[0]
  n_pad = _pad_nodes(n)
  garbage_row = n_pad - 8
  ei = edge_index.astype(jnp.int32)
  e = ei.shape[1]
  n_chunks = e // CHUNK
  nw = NC * NS

  rounds = -(-n_chunks // nw)
  rounds = ((rounds + 3) // 4) * 4
  pad = rounds * nw * CHUNK - e

  def to_worker_blocks(flat, pad_values):
    padded = jnp.concatenate([flat, pad_values])
    return padded.reshape(rounds, nw, CHUNK).transpose(1, 0, 2)

  pad_zero = jnp.zeros((pad,), jnp.int32)
  pad_spread = n + (jnp.arange(pad, dtype=jnp.int32) % (n_pad - n - 8))
  cols3d = to_worker_blocks(ei[1], pad_spread)

  degp = _sc_degree(cols3d, n, rounds)[:, :n]
  h = _tc_linear(x, W)
  h2 = _tc_scale(h, degp)
  q = _sc_spmm(h2, ei[1], ei[0], n)[:, :n]
  return _tc_combine(q, degp, h2)


# speedup vs baseline: 2.8469x; 1.1205x over previous
import functools

import jax
import jax.numpy as jnp
from jax import lax
from jax.experimental import pallas as pl
from jax.experimental.pallas import tpu as pltpu
from jax.experimental.pallas import tpu_sc as plsc

NC = 2
NS = 16
LANES = 16
CHUNK = 128


def _pad_nodes(n):
  return ((n + NS * CHUNK - 1) // (NS * CHUNK)) * (NS * CHUNK)


def _sc_degree(cols3d, n_nodes, rounds):
  n_pad = _pad_nodes(n_nodes)
  rows_per_sub = n_pad // NS
  mesh = plsc.VectorSubcoreMesh(core_axis_name="c", subcore_axis_name="s", num_cores=NC, num_subcores=NS)

  @functools.partial(
      pl.kernel,
      out_type=jax.ShapeDtypeStruct((NC, n_pad, LANES), jnp.float32),
      mesh=mesh,
      scratch_types=[
          pltpu.VMEM((rounds, CHUNK), jnp.int32),
          pltpu.VMEM((CHUNK, LANES), jnp.float32),
          pltpu.VMEM((rows_per_sub, LANES), jnp.float32),
          pltpu.VMEM_SHARED((n_pad, LANES), jnp.float32),
      ],
      compiler_params=pltpu.CompilerParams(use_tc_tiling_on_sc=False),
  )
  def deg_kernel(cols_hbm, out_hbm, idx_v, ones_v, zero_v, acc_sh):
    c = lax.axis_index("c")
    s = lax.axis_index("s")
    wid = s * NC + c

    zero16 = jnp.zeros((LANES,), jnp.float32)
    one_row = jnp.where(lax.iota(jnp.int32, LANES) == 0, 1.0, 0.0)

    @pl.loop(0, rows_per_sub)
    def _(r):
      zero_v[r, :] = zero16

    @pl.loop(0, CHUNK)
    def _(r):
      ones_v[r, :] = one_row

    pltpu.sync_copy(zero_v, acc_sh.at[pl.ds(s * rows_per_sub, rows_per_sub)])
    pltpu.sync_copy(cols_hbm.at[wid], idx_v)
    plsc.subcore_barrier()

    @pl.loop(0, rounds)
    def _(k):
      pltpu.sync_copy(ones_v, acc_sh.at[idx_v.at[k]], add=True)

    plsc.subcore_barrier()
    pltpu.sync_copy(
        acc_sh.at[pl.ds(s * rows_per_sub, rows_per_sub)],
        out_hbm.at[c, pl.ds(s * rows_per_sub, rows_per_sub)],
    )

  return deg_kernel(cols3d)


def _sc_spmm(h2, src, dst, n_nodes):
  e = src.shape[0]
  n_chunks = e // CHUNK
  d = h2.shape[1]
  n_pad = _pad_nodes(n_nodes)
  rows_per_sub = n_pad // NS
  zrows = CHUNK
  mesh = plsc.VectorSubcoreMesh(core_axis_name="c", subcore_axis_name="s", num_cores=NC, num_subcores=NS)

  @functools.partial(
      pl.kernel,
      out_type=jax.ShapeDtypeStruct((NC, n_pad, d), jnp.float32),
      mesh=mesh,
      scratch_types=[
          pltpu.VMEM((CHUNK,), jnp.int32),
          pltpu.VMEM((CHUNK,), jnp.int32),
          pltpu.VMEM((CHUNK,), jnp.int32),
          pltpu.VMEM((CHUNK,), jnp.int32),
          pltpu.VMEM((CHUNK, d), jnp.float32),
          pltpu.VMEM((CHUNK, d), jnp.float32),
          pltpu.VMEM_SHARED((n_pad, d), jnp.float32),
          pltpu.SemaphoreType.DMA,
          pltpu.SemaphoreType.DMA,
          pltpu.SemaphoreType.DMA,
          pltpu.SemaphoreType.DMA,
      ],
  )
  def spmm_kernel(h2_hbm, src_hbm, dst_hbm, out_hbm, sidx0, didx0, sidx1,
                  didx1, gbuf0, gbuf1, acc_sh, sem0, sem1, isem0, isem1):
    c = lax.axis_index("c")
    s = lax.axis_index("s")
    wid = s * NC + c

    zero16 = jnp.zeros((LANES,), jnp.float32)

    @pl.loop(0, zrows)
    def _(r):
      @pl.loop(0, d, step=LANES)
      def _(j):
        gbuf0[r, pl.ds(j, LANES)] = zero16

    @pl.loop(0, rows_per_sub, step=zrows)
    def _(r0):
      pltpu.sync_copy(
          gbuf0.at[pl.ds(0, zrows)],
          acc_sh.at[pl.ds(s * rows_per_sub + r0, zrows)],
      )

    plsc.subcore_barrier()

    nw = NC * NS
    max_rounds = (n_chunks + nw - 1) // nw
    max_rounds += max_rounds % 2

    def start_idx(g, sidx, didx, isem):
      pltpu.make_async_copy(src_hbm.at[pl.ds(g * CHUNK, CHUNK)], sidx,
                            isem).start()
      pltpu.make_async_copy(dst_hbm.at[pl.ds(g * CHUNK, CHUNK)], didx,
                            isem).start()

    def start_gather(sidx, didx, gbuf, sem, isem):
      pltpu.make_async_copy(src_hbm.at[pl.ds(0, CHUNK)], sidx, isem).wait()
      pltpu.make_async_copy(dst_hbm.at[pl.ds(0, CHUNK)], didx, isem).wait()
      pltpu.make_async_copy(h2_hbm.at[sidx], gbuf, sem).start()

    def drain(didx, gbuf, sem):
      pltpu.make_async_copy(h2_hbm.at[sidx0], gbuf, sem).wait()
      pltpu.sync_copy(gbuf, acc_sh.at[didx], add=True)

    @pl.when(wid < n_chunks)
    def _():
      start_idx(wid, sidx0, didx0, isem0)

    @pl.when(wid + nw < n_chunks)
    def _():
      start_idx(wid + nw, sidx1, didx1, isem1)

    @pl.when(wid < n_chunks)
    def _():
      start_gather(sidx0, didx0, gbuf0, sem0, isem0)

    @pl.loop(0, max_rounds, step=2)
    def _(k):
      g0 = wid + k * nw
      g1 = g0 + nw
      g2 = g1 + nw
      g3 = g2 + nw

      @pl.when(g1 < n_chunks)
      def _():
        start_gather(sidx1, didx1, gbuf1, sem1, isem1)

      @pl.when(g0 < n_chunks)
      def _():
        drain(didx0, gbuf0, sem0)

      @pl.when(g2 < n_chunks)
      def _():
        start_idx(g2, sidx0, didx0, isem0)
        start_gather(sidx0, didx0, gbuf0, sem0, isem0)

      @pl.when(g1 < n_chunks)
      def _():
        drain(didx1, gbuf1, sem1)

      @pl.when(g3 < n_chunks)
      def _():
        start_idx(g3, sidx1, didx1, isem1)

    plsc.subcore_barrier()
    pltpu.sync_copy(
        acc_sh.at[pl.ds(s * rows_per_sub, rows_per_sub)],
        out_hbm.at[c, pl.ds(s * rows_per_sub, rows_per_sub)],
    )

  return spmm_kernel(h2, src, dst)


def _tc_linear(x, w):
  n, d_in = x.shape
  d_out = w.shape[0]
  bm = 1000

  def body(x_ref, w_ref, o_ref):
    o_ref[...] = lax.dot_general(
        x_ref[...], w_ref[...],
        (((1,), (1,)), ((), ())),
        precision=lax.Precision.HIGHEST,
    )

  return pl.pallas_call(
      body,
      grid=(n // bm,),
      in_specs=[
          pl.BlockSpec((bm, d_in), lambda i: (i, 0)),
          pl.BlockSpec((d_out, d_in), lambda i: (0, 0)),
      ],
      out_specs=pl.BlockSpec((bm, d_out), lambda i: (i, 0)),
      out_shape=jax.ShapeDtypeStruct((n, d_out), jnp.float32),
  )(x, w)


def _tc_scale(h, degp):
  n, d = h.shape
  bm = 1000

  def body(h_ref, dp_ref, o_ref):
    deg = 1.0 + dp_ref[0, :, 0] + dp_ref[1, :, 0]
    o_ref[...] = h_ref[...] * lax.rsqrt(deg)[:, None]

  return pl.pallas_call(
      body,
      grid=(n // bm,),
      in_specs=[
          pl.BlockSpec((bm, d), lambda i: (i, 0)),
          pl.BlockSpec((NC, bm, LANES), lambda i: (0, i, 0)),
      ],
      out_specs=pl.BlockSpec((bm, d), lambda i: (i, 0)),
      out_shape=jax.ShapeDtypeStruct((n, d), jnp.float32),
  )(h, degp)


def _tc_combine(q, degp, h2):
  n, d = h2.shape
  bm = 1000

  def body(q_ref, dp_ref, h2_ref, o_ref):
    deg = 1.0 + dp_ref[0, :, 0] + dp_ref[1, :, 0]
    agg = q_ref[0] + q_ref[1] + h2_ref[...]
    o_ref[...] = jnp.maximum(agg * lax.rsqrt(deg)[:, None], 0.0)

  return pl.pallas_call(
      body,
      grid=(n // bm,),
      in_specs=[
          pl.BlockSpec((NC, bm, d), lambda i: (0, i, 0)),
          pl.BlockSpec((NC, bm, LANES), lambda i: (0, i, 0)),
          pl.BlockSpec((bm, d), lambda i: (i, 0)),
      ],
      out_specs=pl.BlockSpec((bm, d), lambda i: (i, 0)),
      out_shape=jax.ShapeDtypeStruct((n, d), jnp.float32),
  )(q, degp, h2)


def kernel(x, edge_index, W):
  n = x.shape[0]
  n_pad = _pad_nodes(n)
  garbage_row = n_pad - 8
  ei = edge_index.astype(jnp.int32)
  e = ei.shape[1]
  n_chunks = e // CHUNK
  nw = NC * NS

  rounds = -(-n_chunks // nw)
  rounds = ((rounds + 3) // 4) * 4
  pad = rounds * nw * CHUNK - e

  def to_worker_blocks(flat, pad_values):
    padded = jnp.concatenate([flat, pad_values])
    return padded.reshape(rounds, nw, CHUNK).transpose(1, 0, 2)

  pad_zero = jnp.zeros((pad,), jnp.int32)
  pad_spread = n + (jnp.arange(pad, dtype=jnp.int32) % (n_pad - n - 8))
  cols3d = to_worker_blocks(ei[1], pad_spread)

  degp = _sc_degree(cols3d, n, rounds)[:, :n]
  h = _tc_linear(x, W)
  h2 = _tc_scale(h, degp)
  q = _sc_spmm(h2, ei[1], ei[0], n)[:, :n]
  return _tc_combine(q, degp, h2)


# speedup vs baseline: 2.9971x; 1.0528x over previous
import functools

import jax
import jax.numpy as jnp
from jax import lax
from jax.experimental import pallas as pl
from jax.experimental.pallas import tpu as pltpu
from jax.experimental.pallas import tpu_sc as plsc

NC = 2
NS = 16
LANES = 16
CHUNK = 128


def _pad_nodes(n):
  return ((n + NS * CHUNK - 1) // (NS * CHUNK)) * (NS * CHUNK)


def _sc_degree(cols3d, n_nodes, rounds):
  n_pad = _pad_nodes(n_nodes)
  rows_per_sub = n_pad // NS
  mesh = plsc.VectorSubcoreMesh(core_axis_name="c", subcore_axis_name="s", num_cores=NC, num_subcores=NS)

  @functools.partial(
      pl.kernel,
      out_type=jax.ShapeDtypeStruct((NC, n_pad, LANES), jnp.float32),
      mesh=mesh,
      scratch_types=[
          pltpu.VMEM((rounds, CHUNK), jnp.int32),
          pltpu.VMEM((CHUNK, LANES), jnp.float32),
          pltpu.VMEM((rows_per_sub, LANES), jnp.float32),
          pltpu.VMEM_SHARED((n_pad, LANES), jnp.float32),
      ],
      compiler_params=pltpu.CompilerParams(use_tc_tiling_on_sc=False),
  )
  def deg_kernel(cols_hbm, out_hbm, idx_v, ones_v, zero_v, acc_sh):
    c = lax.axis_index("c")
    s = lax.axis_index("s")
    wid = s * NC + c

    zero16 = jnp.zeros((LANES,), jnp.float32)
    one_row = jnp.where(lax.iota(jnp.int32, LANES) == 0, 1.0, 0.0)

    @pl.loop(0, rows_per_sub)
    def _(r):
      zero_v[r, :] = zero16

    @pl.loop(0, CHUNK)
    def _(r):
      ones_v[r, :] = one_row

    pltpu.sync_copy(zero_v, acc_sh.at[pl.ds(s * rows_per_sub, rows_per_sub)])
    pltpu.sync_copy(cols_hbm.at[wid], idx_v)
    plsc.subcore_barrier()

    @pl.loop(0, rounds)
    def _(k):
      pltpu.sync_copy(ones_v, acc_sh.at[idx_v.at[k]], add=True)

    plsc.subcore_barrier()
    pltpu.sync_copy(
        acc_sh.at[pl.ds(s * rows_per_sub, rows_per_sub)],
        out_hbm.at[c, pl.ds(s * rows_per_sub, rows_per_sub)],
    )

  return deg_kernel(cols3d)


def _sc_spmm(h2, src, dst, n_nodes):
  e = src.shape[0]
  n_chunks = e // CHUNK
  d = h2.shape[1]
  n_pad = _pad_nodes(n_nodes)
  rows_per_sub = n_pad // NS
  zrows = CHUNK
  mesh = plsc.VectorSubcoreMesh(core_axis_name="c", subcore_axis_name="s", num_cores=NC, num_subcores=NS)

  @functools.partial(
      pl.kernel,
      out_type=jax.ShapeDtypeStruct((NC, n_pad, d), jnp.float32),
      mesh=mesh,
      scratch_types=[
          pltpu.VMEM((CHUNK,), jnp.int32),
          pltpu.VMEM((CHUNK,), jnp.int32),
          pltpu.VMEM((CHUNK,), jnp.int32),
          pltpu.VMEM((CHUNK,), jnp.int32),
          pltpu.VMEM((CHUNK, d), jnp.float32),
          pltpu.VMEM((CHUNK, d), jnp.float32),
          pltpu.VMEM_SHARED((n_pad, d), jnp.float32),
          pltpu.SemaphoreType.DMA,
          pltpu.SemaphoreType.DMA,
          pltpu.SemaphoreType.DMA,
          pltpu.SemaphoreType.DMA,
      ],
  )
  def spmm_kernel(h2_hbm, src_hbm, dst_hbm, out_hbm, sidx0, didx0, sidx1,
                  didx1, gbuf0, gbuf1, acc_sh, sem0, sem1, isem0, isem1):
    c = lax.axis_index("c")
    s = lax.axis_index("s")
    wid = s * NC + c

    zero16 = jnp.zeros((LANES,), jnp.float32)

    @pl.loop(0, zrows)
    def _(r):
      @pl.loop(0, d, step=LANES)
      def _(j):
        gbuf0[r, pl.ds(j, LANES)] = zero16

    @pl.loop(0, rows_per_sub, step=zrows)
    def _(r0):
      pltpu.sync_copy(
          gbuf0.at[pl.ds(0, zrows)],
          acc_sh.at[pl.ds(s * rows_per_sub + r0, zrows)],
      )

    plsc.subcore_barrier()

    nw = NC * NS
    max_rounds = (n_chunks + nw - 1) // nw
    max_rounds += max_rounds % 2

    def start_idx(g, sidx, didx, isem):
      pltpu.make_async_copy(src_hbm.at[pl.ds(g * CHUNK, CHUNK)], sidx,
                            isem).start()
      pltpu.make_async_copy(dst_hbm.at[pl.ds(g * CHUNK, CHUNK)], didx,
                            isem).start()

    def start_gather(sidx, didx, gbuf, sem, isem):
      pltpu.make_async_copy(src_hbm.at[pl.ds(0, CHUNK)], sidx, isem).wait()
      pltpu.make_async_copy(dst_hbm.at[pl.ds(0, CHUNK)], didx, isem).wait()
      pltpu.make_async_copy(h2_hbm.at[sidx], gbuf, sem).start()

    def drain(didx, gbuf, sem):
      pltpu.make_async_copy(h2_hbm.at[sidx0], gbuf, sem).wait()
      pltpu.sync_copy(gbuf, acc_sh.at[didx], add=True)

    @pl.when(wid < n_chunks)
    def _():
      start_idx(wid, sidx0, didx0, isem0)

    @pl.when(wid + nw < n_chunks)
    def _():
      start_idx(wid + nw, sidx1, didx1, isem1)

    @pl.when(wid < n_chunks)
    def _():
      start_gather(sidx0, didx0, gbuf0, sem0, isem0)

    @pl.loop(0, max_rounds, step=2)
    def _(k):
      g0 = wid + k * nw
      g1 = g0 + nw
      g2 = g1 + nw
      g3 = g2 + nw

      @pl.when(g1 < n_chunks)
      def _():
        start_gather(sidx1, didx1, gbuf1, sem1, isem1)

      @pl.when(g0 < n_chunks)
      def _():
        drain(didx0, gbuf0, sem0)

      @pl.when(g2 < n_chunks)
      def _():
        start_idx(g2, sidx0, didx0, isem0)
        start_gather(sidx0, didx0, gbuf0, sem0, isem0)

      @pl.when(g1 < n_chunks)
      def _():
        drain(didx1, gbuf1, sem1)

      @pl.when(g3 < n_chunks)
      def _():
        start_idx(g3, sidx1, didx1, isem1)

    plsc.subcore_barrier()
    pltpu.sync_copy(
        acc_sh.at[pl.ds(s * rows_per_sub, rows_per_sub)],
        out_hbm.at[c, pl.ds(s * rows_per_sub, rows_per_sub)],
    )

  return spmm_kernel(h2, src, dst)


def _tc_linear_scale(x, w, degp):
  n, d_in = x.shape
  d_out = w.shape[0]
  bm = 1000

  def body(x_ref, w_ref, dp_ref, o_ref):
    h = lax.dot_general(
        x_ref[...], w_ref[...],
        (((1,), (1,)), ((), ())),
        precision=lax.Precision.HIGHEST,
    )
    deg = 1.0 + dp_ref[0, :, 0] + dp_ref[1, :, 0]
    o_ref[...] = h * lax.rsqrt(deg)[:, None]

  return pl.pallas_call(
      body,
      grid=(n // bm,),
      in_specs=[
          pl.BlockSpec((bm, d_in), lambda i: (i, 0)),
          pl.BlockSpec((d_out, d_in), lambda i: (0, 0)),
          pl.BlockSpec((NC, bm, LANES), lambda i: (0, i, 0)),
      ],
      out_specs=pl.BlockSpec((bm, d_out), lambda i: (i, 0)),
      out_shape=jax.ShapeDtypeStruct((n, d_out), jnp.float32),
  )(x, w, degp)


def _tc_combine(q, degp, h2):
  n, d = h2.shape
  bm = 1000

  def body(q_ref, dp_ref, h2_ref, o_ref):
    deg = 1.0 + dp_ref[0, :, 0] + dp_ref[1, :, 0]
    agg = q_ref[0] + q_ref[1] + h2_ref[...]
    o_ref[...] = jnp.maximum(agg * lax.rsqrt(deg)[:, None], 0.0)

  return pl.pallas_call(
      body,
      grid=(n // bm,),
      in_specs=[
          pl.BlockSpec((NC, bm, d), lambda i: (0, i, 0)),
          pl.BlockSpec((NC, bm, LANES), lambda i: (0, i, 0)),
          pl.BlockSpec((bm, d), lambda i: (i, 0)),
      ],
      out_specs=pl.BlockSpec((bm, d), lambda i: (i, 0)),
      out_shape=jax.ShapeDtypeStruct((n, d), jnp.float32),
  )(q, degp, h2)


def kernel(x, edge_index, W):
  n = x.shape[0]
  n_pad = _pad_nodes(n)
  garbage_row = n_pad - 8
  ei = edge_index.astype(jnp.int32)
  e = ei.shape[1]
  n_chunks = e // CHUNK
  nw = NC * NS

  rounds = -(-n_chunks // nw)
  rounds = ((rounds + 3) // 4) * 4
  pad = rounds * nw * CHUNK - e

  def to_worker_blocks(flat, pad_values):
    padded = jnp.concatenate([flat, pad_values])
    return padded.reshape(rounds, nw, CHUNK).transpose(1, 0, 2)

  pad_zero = jnp.zeros((pad,), jnp.int32)
  pad_spread = n + (jnp.arange(pad, dtype=jnp.int32) % (n_pad - n - 8))
  cols3d = to_worker_blocks(ei[1], pad_spread)

  degp = _sc_degree(cols3d, n, rounds)
  h2 = _tc_linear_scale(x, W, degp)
  q = _sc_spmm(h2, ei[1], ei[0], n)
  return _tc_combine(q, degp, h2)
